# baseline (speedup 1.0000x reference)
import functools

import jax
import jax.numpy as jnp
from jax import lax
from jax.experimental import pallas as pl
from jax.experimental.pallas import tpu as pltpu
from jax.experimental.pallas import tpu_sc as plsc

_THRESH = 0.7
_MIN_KEPT = 10000

_ROWS = 8192
_COLS = 512
_N = _ROWS * _COLS

_NC = 2
_NS = 16
_LANES = 16
_NW = _NC * _NS
_PER_W = _N // _NW
_CH = 16384
_NCHUNK = _PER_W // _CH

_T_HI = 0.8472978603872037


def _chunk_reduce(x_ref, y_ref, s_acc, n_acc):
    def step(i, carry):
        s, n = carry
        x = x_ref[pl.ds(i * _LANES, _LANES)]
        y = y_ref[pl.ds(i * _LANES, _LANES)]
        t = jnp.abs(x)
        u = jnp.exp(-t)
        w = u / (2.0 + u)
        w2 = w * w
        sp = (2.0 * w) * (1.0 + w2 * (1.0 / 3.0 + w2 * (0.2 + w2 * (1.0 / 7.0))))
        per = jnp.maximum(x, 0.0) - x * y + sp
        kept = jnp.where(y == 1.0, x <= _T_HI, x >= -_T_HI)
        s = s + jnp.where(kept, per, 0.0)
        n = n + jnp.where(kept, 1.0, 0.0)
        return s, n

    return lax.fori_loop(0, _CH // _LANES, step, (s_acc, n_acc))


def _sc_sums(x_flat, y_flat):
    mesh = plsc.VectorSubcoreMesh(core_axis_name="c", subcore_axis_name="s")

    @functools.partial(
        pl.kernel,
        out_type=[
            jax.ShapeDtypeStruct((_NW, _LANES), jnp.float32),
            jax.ShapeDtypeStruct((_NW, _LANES), jnp.float32),
        ],
        mesh=mesh,
        scratch_types=[
            pltpu.VMEM((_CH,), jnp.float32),
            pltpu.VMEM((_CH,), jnp.float32),
            pltpu.VMEM((_CH,), jnp.float32),
            pltpu.VMEM((_CH,), jnp.float32),
            pltpu.VMEM((_LANES,), jnp.float32),
            pltpu.VMEM((_LANES,), jnp.float32),
            pltpu.SemaphoreType.DMA,
            pltpu.SemaphoreType.DMA,
            pltpu.SemaphoreType.DMA,
            pltpu.SemaphoreType.DMA,
        ],
    )
    def k(x_hbm, y_hbm, s_out, n_out, xb0, yb0, xb1, yb1, sv, nv,
          sx0, sy0, sx1, sy1):
        wid = lax.axis_index("s") * _NC + lax.axis_index("c")
        base = wid * _PER_W
        xbufs = (xb0, xb1)
        ybufs = (yb0, yb1)
        sxs = (sx0, sx1)
        sys_ = (sy0, sy1)

        handles = {
            0: (
                pltpu.async_copy(x_hbm.at[pl.ds(base, _CH)], xb0, sx0),
                pltpu.async_copy(y_hbm.at[pl.ds(base, _CH)], yb0, sy0),
            )
        }
        s_acc = jnp.zeros((_LANES,), jnp.float32)
        n_acc = jnp.zeros((_LANES,), jnp.float32)
        for ch in range(_NCHUNK):
            cur = ch % 2
            if ch + 1 < _NCHUNK:
                nxt = (ch + 1) % 2
                off = base + (ch + 1) * _CH
                handles[ch + 1] = (
                    pltpu.async_copy(x_hbm.at[pl.ds(off, _CH)], xbufs[nxt], sxs[nxt]),
                    pltpu.async_copy(y_hbm.at[pl.ds(off, _CH)], ybufs[nxt], sys_[nxt]),
                )
            hx, hy = handles.pop(ch)
            hx.wait()
            hy.wait()
            s_acc, n_acc = _chunk_reduce(xbufs[cur], ybufs[cur], s_acc, n_acc)

        sv[...] = s_acc
        nv[...] = n_acc
        pltpu.sync_copy(sv, s_out.at[wid])
        pltpu.sync_copy(nv, n_out.at[wid])

    return k(x_flat, y_flat)


def _bce_terms(x, y):
    p = jax.nn.sigmoid(x)
    kept = ((y == 1.0) & (p <= _THRESH)) | ((y == 0.0) & (p >= 1.0 - _THRESH))
    per = jnp.maximum(x, 0.0) - x * y + jnp.log1p(jnp.exp(-jnp.abs(x)))
    return per, kept.astype(jnp.float32)


def _fallback_loss(ops):
    x, y, s, n = ops
    p = jax.nn.sigmoid(x)
    per, kf = _bce_terms(x, y)
    h = jnp.abs(p - 0.5).reshape(-1)
    _, idx = lax.top_k(-h, _MIN_KEPT)
    extra = 1.0 - kf.reshape(-1)[idx]
    s2 = s + jnp.sum(per.reshape(-1)[idx] * extra)
    n2 = n + jnp.sum(extra)
    return s2 / jnp.maximum(n2, 1.0)


def _main_loss(ops):
    _, _, s, n = ops
    return s / jnp.maximum(n, 1.0)


def kernel(input, target):
    x = input.reshape(_ROWS, _COLS)
    y = target.reshape(_ROWS, _COLS).astype(jnp.float32)
    s_p, n_p = _sc_sums(input.reshape(_N), y.reshape(_N))
    s = jnp.sum(s_p)
    n = jnp.sum(n_p)
    return lax.cond(n < _MIN_KEPT, _fallback_loss, _main_loss, (x, y, s, n))


# speedup vs baseline: 1.1982x; 1.1982x over previous
import functools

import jax
import jax.numpy as jnp
from jax import lax
from jax.experimental import pallas as pl
from jax.experimental.pallas import tpu as pltpu
from jax.experimental.pallas import tpu_sc as plsc

_THRESH = 0.7
_MIN_KEPT = 10000

_ROWS = 8192
_COLS = 512
_N = _ROWS * _COLS

_NC = 2
_NS = 16
_LANES = 16
_NW = _NC * _NS
_PER_W = _N // _NW
_CH = 16384
_NCHUNK = _PER_W // _CH

_T_HI = 0.8472978603872037


_L1P_C0 = 0.9996203753455163
_L1P_C1 = -0.48664306404532276
_L1P_C2 = 0.25462220684705594
_L1P_C3 = -0.07473614766179369


def _chunk_reduce(x_ref, y_ref, s_acc, n_acc):
    def step(i, carry):
        s, n = carry
        x = x_ref[pl.ds(i * _LANES, _LANES)]
        y = y_ref[pl.ds(i * _LANES, _LANES)]
        ybits = lax.bitcast_convert_type(y, jnp.uint32)
        xbits = lax.bitcast_convert_type(x, jnp.uint32)
        z = lax.bitcast_convert_type(xbits ^ (ybits << 8), jnp.float32)
        u = jnp.exp(-jnp.abs(z))
        sp = u * (_L1P_C0 + u * (_L1P_C1 + u * (_L1P_C2 + u * _L1P_C3)))
        kept = z >= -_T_HI
        per = jnp.maximum(z, 0.0) + sp
        s = s + jnp.where(kept, per, 0.0)
        n = n + jnp.where(kept, 1.0, 0.0)
        return s, n

    return lax.fori_loop(0, _CH // _LANES, step, (s_acc, n_acc))


def _sc_sums(x_flat, y_flat):
    mesh = plsc.VectorSubcoreMesh(core_axis_name="c", subcore_axis_name="s")

    @functools.partial(
        pl.kernel,
        out_type=[
            jax.ShapeDtypeStruct((_NW, _LANES), jnp.float32),
            jax.ShapeDtypeStruct((_NW, _LANES), jnp.float32),
        ],
        mesh=mesh,
        scratch_types=[
            pltpu.VMEM((_CH,), jnp.float32),
            pltpu.VMEM((_CH,), jnp.float32),
            pltpu.VMEM((_CH,), jnp.float32),
            pltpu.VMEM((_CH,), jnp.float32),
            pltpu.VMEM((_LANES,), jnp.float32),
            pltpu.VMEM((_LANES,), jnp.float32),
            pltpu.SemaphoreType.DMA,
            pltpu.SemaphoreType.DMA,
            pltpu.SemaphoreType.DMA,
            pltpu.SemaphoreType.DMA,
        ],
    )
    def k(x_hbm, y_hbm, s_out, n_out, xb0, yb0, xb1, yb1, sv, nv,
          sx0, sy0, sx1, sy1):
        wid = lax.axis_index("s") * _NC + lax.axis_index("c")
        base = wid * _PER_W
        xbufs = (xb0, xb1)
        ybufs = (yb0, yb1)
        sxs = (sx0, sx1)
        sys_ = (sy0, sy1)

        handles = {
            0: (
                pltpu.async_copy(x_hbm.at[pl.ds(base, _CH)], xb0, sx0),
                pltpu.async_copy(y_hbm.at[pl.ds(base, _CH)], yb0, sy0),
            )
        }
        s_acc = jnp.zeros((_LANES,), jnp.float32)
        n_acc = jnp.zeros((_LANES,), jnp.float32)
        for ch in range(_NCHUNK):
            cur = ch % 2
            if ch + 1 < _NCHUNK:
                nxt = (ch + 1) % 2
                off = base + (ch + 1) * _CH
                handles[ch + 1] = (
                    pltpu.async_copy(x_hbm.at[pl.ds(off, _CH)], xbufs[nxt], sxs[nxt]),
                    pltpu.async_copy(y_hbm.at[pl.ds(off, _CH)], ybufs[nxt], sys_[nxt]),
                )
            hx, hy = handles.pop(ch)
            hx.wait()
            hy.wait()
            s_acc, n_acc = _chunk_reduce(xbufs[cur], ybufs[cur], s_acc, n_acc)

        sv[...] = s_acc
        nv[...] = n_acc
        pltpu.sync_copy(sv, s_out.at[wid])
        pltpu.sync_copy(nv, n_out.at[wid])

    return k(x_flat, y_flat)


def _bce_terms(x, y):
    p = jax.nn.sigmoid(x)
    kept = ((y == 1.0) & (p <= _THRESH)) | ((y == 0.0) & (p >= 1.0 - _THRESH))
    per = jnp.maximum(x, 0.0) - x * y + jnp.log1p(jnp.exp(-jnp.abs(x)))
    return per, kept.astype(jnp.float32)


def _fallback_loss(ops):
    x, y, s, n = ops
    p = jax.nn.sigmoid(x)
    per, kf = _bce_terms(x, y)
    h = jnp.abs(p - 0.5).reshape(-1)
    _, idx = lax.top_k(-h, _MIN_KEPT)
    extra = 1.0 - kf.reshape(-1)[idx]
    s2 = s + jnp.sum(per.reshape(-1)[idx] * extra)
    n2 = n + jnp.sum(extra)
    return s2 / jnp.maximum(n2, 1.0)


def _main_loss(ops):
    _, _, s, n = ops
    return s / jnp.maximum(n, 1.0)


def kernel(input, target):
    x = input.reshape(_ROWS, _COLS)
    y = target.reshape(_ROWS, _COLS).astype(jnp.float32)
    s_p, n_p = _sc_sums(input.reshape(_N), y.reshape(_N))
    s = jnp.sum(s_p)
    n = jnp.sum(n_p)
    return lax.cond(n < _MIN_KEPT, _fallback_loss, _main_loss, (x, y, s, n))


# speedup vs baseline: 1.4797x; 1.2349x over previous
import functools

import jax
import jax.numpy as jnp
from jax import lax
from jax.experimental import pallas as pl
from jax.experimental.pallas import tpu as pltpu
from jax.experimental.pallas import tpu_sc as plsc

_THRESH = 0.7
_MIN_KEPT = 10000

_ROWS = 8192
_COLS = 512
_N = _ROWS * _COLS

_R_SC = 3072
_R_TC = _ROWS - _R_SC
_N_SC = _R_SC * _COLS

_BLOCK_ROWS = 1024
_TC_OFF = _R_SC // _BLOCK_ROWS
_TC_GRID = _R_TC // _BLOCK_ROWS

_NC = 2
_NS = 16
_LANES = 16
_NW = _NC * _NS
_PER_W = _N_SC // _NW
_CH = 16384
_NCHUNK = _PER_W // _CH

_T_HI = 0.8472978603872037

_L1P_C0 = 0.9996203753455163
_L1P_C1 = -0.48664306404532276
_L1P_C2 = 0.25462220684705594
_L1P_C3 = -0.07473614766179369


def _chunk_reduce(x_ref, y_ref, s_acc, n_acc):
    def step(i, carry):
        s, n = carry
        x = x_ref[pl.ds(i * _LANES, _LANES)]
        y = y_ref[pl.ds(i * _LANES, _LANES)]
        ybits = lax.bitcast_convert_type(y, jnp.uint32)
        xbits = lax.bitcast_convert_type(x, jnp.uint32)
        z = lax.bitcast_convert_type(xbits ^ (ybits << 8), jnp.float32)
        u = jnp.exp(-jnp.abs(z))
        sp = u * (_L1P_C0 + u * (_L1P_C1 + u * (_L1P_C2 + u * _L1P_C3)))
        kept = z >= -_T_HI
        per = jnp.maximum(z, 0.0) + sp
        s = s + jnp.where(kept, per, 0.0)
        n = n + jnp.where(kept, 1.0, 0.0)
        return s, n

    return lax.fori_loop(0, _CH // _LANES, step, (s_acc, n_acc))


def _sc_sums(x_flat, y_flat):
    mesh = plsc.VectorSubcoreMesh(core_axis_name="c", subcore_axis_name="s")

    @functools.partial(
        pl.kernel,
        out_type=[
            jax.ShapeDtypeStruct((_NW, _LANES), jnp.float32),
            jax.ShapeDtypeStruct((_NW, _LANES), jnp.float32),
        ],
        mesh=mesh,
        scratch_types=[
            pltpu.VMEM((_CH,), jnp.float32),
            pltpu.VMEM((_CH,), jnp.float32),
            pltpu.VMEM((_CH,), jnp.float32),
            pltpu.VMEM((_CH,), jnp.float32),
            pltpu.VMEM((_LANES,), jnp.float32),
            pltpu.VMEM((_LANES,), jnp.float32),
            pltpu.SemaphoreType.DMA,
            pltpu.SemaphoreType.DMA,
            pltpu.SemaphoreType.DMA,
            pltpu.SemaphoreType.DMA,
        ],
    )
    def k(x_hbm, y_hbm, s_out, n_out, xb0, yb0, xb1, yb1, sv, nv,
          sx0, sy0, sx1, sy1):
        wid = lax.axis_index("s") * _NC + lax.axis_index("c")
        base = wid * _PER_W
        xbufs = (xb0, xb1)
        ybufs = (yb0, yb1)
        sxs = (sx0, sx1)
        sys_ = (sy0, sy1)

        handles = {
            0: (
                pltpu.async_copy(x_hbm.at[pl.ds(base, _CH)], xb0, sx0),
                pltpu.async_copy(y_hbm.at[pl.ds(base, _CH)], yb0, sy0),
            )
        }
        s_acc = jnp.zeros((_LANES,), jnp.float32)
        n_acc = jnp.zeros((_LANES,), jnp.float32)
        for ch in range(_NCHUNK):
            cur = ch % 2
            if ch + 1 < _NCHUNK:
                nxt = (ch + 1) % 2
                off = base + (ch + 1) * _CH
                handles[ch + 1] = (
                    pltpu.async_copy(x_hbm.at[pl.ds(off, _CH)], xbufs[nxt], sxs[nxt]),
                    pltpu.async_copy(y_hbm.at[pl.ds(off, _CH)], ybufs[nxt], sys_[nxt]),
                )
            hx, hy = handles.pop(ch)
            hx.wait()
            hy.wait()
            s_acc, n_acc = _chunk_reduce(xbufs[cur], ybufs[cur], s_acc, n_acc)

        sv[...] = s_acc
        nv[...] = n_acc
        pltpu.sync_copy(sv, s_out.at[wid])
        pltpu.sync_copy(nv, n_out.at[wid])

    return k(x_flat, y_flat)


def _tc_body(x_ref, y_ref, s_ref, n_ref):
    i = pl.program_id(0)
    x = x_ref[...]
    y = y_ref[...]
    z = x * (1.0 - 2.0 * y)
    kept = (z >= -_T_HI).astype(jnp.float32)
    per = jnp.maximum(z, 0.0) + jnp.log1p(jnp.exp(-jnp.abs(z)))

    @pl.when(i == 0)
    def _init():
        s_ref[...] = jnp.zeros((1, 1), jnp.float32)
        n_ref[...] = jnp.zeros((1, 1), jnp.float32)

    s_ref[...] += jnp.sum(per * kept).reshape(1, 1)
    n_ref[...] += jnp.sum(kept).reshape(1, 1)


def _tc_sums(x, y):
    in_spec = pl.BlockSpec((_BLOCK_ROWS, _COLS), lambda i: (i + _TC_OFF, 0))
    out_spec = pl.BlockSpec((1, 1), lambda i: (0, 0))
    s, n = pl.pallas_call(
        _tc_body,
        grid=(_TC_GRID,),
        in_specs=[in_spec, in_spec],
        out_specs=[out_spec, out_spec],
        out_shape=[
            jax.ShapeDtypeStruct((1, 1), jnp.float32),
            jax.ShapeDtypeStruct((1, 1), jnp.float32),
        ],
    )(x, y)
    return s[0, 0], n[0, 0]


def _bce_terms(x, y):
    p = jax.nn.sigmoid(x)
    kept = ((y == 1.0) & (p <= _THRESH)) | ((y == 0.0) & (p >= 1.0 - _THRESH))
    per = jnp.maximum(x, 0.0) - x * y + jnp.log1p(jnp.exp(-jnp.abs(x)))
    return per, kept.astype(jnp.float32)


def _fallback_loss(ops):
    x, y, s, n = ops
    p = jax.nn.sigmoid(x)
    per, kf = _bce_terms(x, y)
    h = jnp.abs(p - 0.5).reshape(-1)
    _, idx = lax.top_k(-h, _MIN_KEPT)
    extra = 1.0 - kf.reshape(-1)[idx]
    s2 = s + jnp.sum(per.reshape(-1)[idx] * extra)
    n2 = n + jnp.sum(extra)
    return s2 / jnp.maximum(n2, 1.0)


def _main_loss(ops):
    _, _, s, n = ops
    return s / jnp.maximum(n, 1.0)


def kernel(input, target):
    x = input.reshape(_ROWS, _COLS)
    y = target.reshape(_ROWS, _COLS).astype(jnp.float32)
    s_p, n_p = _sc_sums(input.reshape(_N), y.reshape(_N))
    s_tc, n_tc = _tc_sums(x, y)
    s = jnp.sum(s_p) + s_tc
    n = jnp.sum(n_p) + n_tc
    return lax.cond(n < _MIN_KEPT, _fallback_loss, _main_loss, (x, y, s, n))


# speedup vs baseline: 2.5920x; 1.7518x over previous
import functools

import jax
import jax.numpy as jnp
from jax import lax
from jax.experimental import pallas as pl
from jax.experimental.pallas import tpu as pltpu
from jax.experimental.pallas import tpu_sc as plsc

_THRESH = 0.7
_MIN_KEPT = 10000

_ROWS = 8192
_COLS = 512
_N = _ROWS * _COLS

_R_SC = 3072
_R_TC = _ROWS - _R_SC
_N_SC = _R_SC * _COLS

_BLOCK_ROWS = 1024
_TC_OFF = _R_SC // _BLOCK_ROWS
_TC_GRID = _R_TC // _BLOCK_ROWS

_NC = 2
_NS = 16
_LANES = 16
_NW = _NC * _NS
_ROWS_W = _R_SC // _NW
_CHR = 32
_CH = _CHR * _COLS
_NCHUNK = _ROWS_W // _CHR

_T_HI = 0.8472978603872037

_L1P_C0 = 0.9996203753455163
_L1P_C1 = -0.48664306404532276
_L1P_C2 = 0.25462220684705594
_L1P_C3 = -0.07473614766179369


def _chunk_reduce(x_ref, y_ref, s_acc, n_acc):
    def step(i, carry):
        s, n = carry
        r = i >> 5
        c = (i & 31) * _LANES
        x = x_ref[r, pl.ds(c, _LANES)]
        y = y_ref[r, pl.ds(c, _LANES)]
        ybits = lax.bitcast_convert_type(y, jnp.uint32)
        xbits = lax.bitcast_convert_type(x, jnp.uint32)
        z = lax.bitcast_convert_type(xbits ^ (ybits << 8), jnp.float32)
        u = jnp.exp(-jnp.abs(z))
        sp = u * (_L1P_C0 + u * (_L1P_C1 + u * (_L1P_C2 + u * _L1P_C3)))
        kept = z >= -_T_HI
        per = jnp.maximum(z, 0.0) + sp
        s = s + jnp.where(kept, per, 0.0)
        n = n + jnp.where(kept, 1.0, 0.0)
        return s, n

    return lax.fori_loop(0, _CH // _LANES, step, (s_acc, n_acc))


def _sc_sums(x2d, y2d):
    mesh = plsc.VectorSubcoreMesh(core_axis_name="c", subcore_axis_name="s")

    @functools.partial(
        pl.kernel,
        out_type=[
            jax.ShapeDtypeStruct((_NW, _LANES), jnp.float32),
            jax.ShapeDtypeStruct((_NW, _LANES), jnp.float32),
        ],
        mesh=mesh,
        compiler_params=pltpu.CompilerParams(use_tc_tiling_on_sc=True),
        scratch_types=[
            pltpu.VMEM((_CHR, _COLS), jnp.float32),
            pltpu.VMEM((_CHR, _COLS), jnp.float32),
            pltpu.VMEM((_CHR, _COLS), jnp.float32),
            pltpu.VMEM((_CHR, _COLS), jnp.float32),
            pltpu.VMEM((_LANES,), jnp.float32),
            pltpu.VMEM((_LANES,), jnp.float32),
            pltpu.SemaphoreType.DMA,
            pltpu.SemaphoreType.DMA,
            pltpu.SemaphoreType.DMA,
            pltpu.SemaphoreType.DMA,
        ],
    )
    def k(x_hbm, y_hbm, s_out, n_out, xb0, yb0, xb1, yb1, sv, nv,
          sx0, sy0, sx1, sy1):
        wid = lax.axis_index("s") * _NC + lax.axis_index("c")
        base = wid * _ROWS_W
        xbufs = (xb0, xb1)
        ybufs = (yb0, yb1)
        sxs = (sx0, sx1)
        sys_ = (sy0, sy1)

        handles = {
            0: (
                pltpu.async_copy(x_hbm.at[pl.ds(base, _CHR)], xb0, sx0),
                pltpu.async_copy(y_hbm.at[pl.ds(base, _CHR)], yb0, sy0),
            )
        }
        s_acc = jnp.zeros((_LANES,), jnp.float32)
        n_acc = jnp.zeros((_LANES,), jnp.float32)
        for ch in range(_NCHUNK):
            cur = ch % 2
            if ch + 1 < _NCHUNK:
                nxt = (ch + 1) % 2
                off = base + (ch + 1) * _CHR
                handles[ch + 1] = (
                    pltpu.async_copy(x_hbm.at[pl.ds(off, _CHR)], xbufs[nxt], sxs[nxt]),
                    pltpu.async_copy(y_hbm.at[pl.ds(off, _CHR)], ybufs[nxt], sys_[nxt]),
                )
            hx, hy = handles.pop(ch)
            hx.wait()
            hy.wait()
            s_acc, n_acc = _chunk_reduce(xbufs[cur], ybufs[cur], s_acc, n_acc)

        sv[...] = s_acc
        nv[...] = n_acc
        pltpu.sync_copy(sv, s_out.at[wid])
        pltpu.sync_copy(nv, n_out.at[wid])

    return k(x2d, y2d)


def _tc_body(x_ref, y_ref, s_ref, n_ref):
    i = pl.program_id(0)
    x = x_ref[...]
    y = y_ref[...]
    z = x * (1.0 - 2.0 * y)
    kept = (z >= -_T_HI).astype(jnp.float32)
    per = jnp.maximum(z, 0.0) + jnp.log1p(jnp.exp(-jnp.abs(z)))

    @pl.when(i == 0)
    def _init():
        s_ref[...] = jnp.zeros((1, 1), jnp.float32)
        n_ref[...] = jnp.zeros((1, 1), jnp.float32)

    s_ref[...] += jnp.sum(per * kept).reshape(1, 1)
    n_ref[...] += jnp.sum(kept).reshape(1, 1)


def _tc_sums(x, y):
    in_spec = pl.BlockSpec((_BLOCK_ROWS, _COLS), lambda i: (i + _TC_OFF, 0))
    out_spec = pl.BlockSpec((1, 1), lambda i: (0, 0))
    s, n = pl.pallas_call(
        _tc_body,
        grid=(_TC_GRID,),
        in_specs=[in_spec, in_spec],
        out_specs=[out_spec, out_spec],
        out_shape=[
            jax.ShapeDtypeStruct((1, 1), jnp.float32),
            jax.ShapeDtypeStruct((1, 1), jnp.float32),
        ],
    )(x, y)
    return s[0, 0], n[0, 0]


def _bce_terms(x, y):
    p = jax.nn.sigmoid(x)
    kept = ((y == 1.0) & (p <= _THRESH)) | ((y == 0.0) & (p >= 1.0 - _THRESH))
    per = jnp.maximum(x, 0.0) - x * y + jnp.log1p(jnp.exp(-jnp.abs(x)))
    return per, kept.astype(jnp.float32)


def _fallback_loss(ops):
    x, y, s, n = ops
    p = jax.nn.sigmoid(x)
    per, kf = _bce_terms(x, y)
    h = jnp.abs(p - 0.5).reshape(-1)
    _, idx = lax.top_k(-h, _MIN_KEPT)
    extra = 1.0 - kf.reshape(-1)[idx]
    s2 = s + jnp.sum(per.reshape(-1)[idx] * extra)
    n2 = n + jnp.sum(extra)
    return s2 / jnp.maximum(n2, 1.0)


def _main_loss(ops):
    _, _, s, n = ops
    return s / jnp.maximum(n, 1.0)


def kernel(input, target):
    x = input.reshape(_ROWS, _COLS)
    y = target.reshape(_ROWS, _COLS).astype(jnp.float32)
    s_p, n_p = _sc_sums(x, y)
    s_tc, n_tc = _tc_sums(x, y)
    s = jnp.sum(s_p) + s_tc
    n = jnp.sum(n_p) + n_tc
    return lax.cond(n < _MIN_KEPT, _fallback_loss, _main_loss, (x, y, s, n))
